# baseline (speedup 1.0000x reference)
import functools

import jax
import jax.numpy as jnp
from jax import lax
from jax.experimental import pallas as pl
from jax.experimental.pallas import tpu as pltpu
from jax.experimental.pallas import tpu_sc as plsc

_B = 16384
_F = 26
_NW = 32
_RPW = _B // _NW
_CHUNK = _F * _RPW


def _make_kernel():
    mesh = plsc.VectorSubcoreMesh(core_axis_name="c", subcore_axis_name="s")

    @functools.partial(
        pl.kernel,
        mesh=mesh,
        out_type=jax.ShapeDtypeStruct((_B,), jnp.float32),
        compiler_params=pltpu.CompilerParams(needs_layout_passes=False),
        scratch_types=[
            pltpu.VMEM((_CHUNK,), jnp.int32),
            pltpu.VMEM((_CHUNK,), jnp.float32),
            pltpu.VMEM((_RPW,), jnp.float32),
            pltpu.SemaphoreType.DMA,
        ],
    )
    def k(idx_hbm, w_hbm, out_hbm, idx_v, vals_v, out_v, sem):
        wid = lax.axis_index("s") * 2 + lax.axis_index("c")
        pltpu.sync_copy(idx_hbm.at[pl.ds(wid * _CHUNK, _CHUNK)], idx_v)
        pltpu.async_copy(w_hbm.at[idx_v], vals_v, sem).wait()
        lanes = lax.iota(jnp.int32, 16) * _F
        for g in range(_RPW // 16):
            base = g * 16 * _F
            acc = plsc.load_gather(vals_v, [lanes + base])
            for f in range(1, _F):
                acc = acc + plsc.load_gather(vals_v, [lanes + (base + f)])
            out_v[pl.ds(g * 16, 16)] = acc
        pltpu.sync_copy(out_v, out_hbm.at[pl.ds(wid * _RPW, _RPW)])

    return k


_sc_kernel = _make_kernel()


def kernel(inputs, w):
    out = _sc_kernel(inputs.reshape(-1).astype(jnp.int32), w.reshape(-1))
    return out.reshape(_B, 1)


# speedup vs baseline: 1.1384x; 1.1384x over previous
import functools

import jax
import jax.numpy as jnp
from jax import lax
from jax.experimental import pallas as pl
from jax.experimental.pallas import tpu as pltpu
from jax.experimental.pallas import tpu_sc as plsc

_B = 16384
_F = 26
_NW = 32
_RPW = _B // _NW
_CHUNK = _F * _RPW


def _make_kernel():
    mesh = plsc.VectorSubcoreMesh(core_axis_name="c", subcore_axis_name="s")

    @functools.partial(
        pl.kernel,
        mesh=mesh,
        out_type=jax.ShapeDtypeStruct((_B,), jnp.float32),
        compiler_params=pltpu.CompilerParams(needs_layout_passes=False),
        scratch_types=[
            pltpu.VMEM((_CHUNK,), jnp.int32),
            pltpu.VMEM((_CHUNK,), jnp.float32),
            pltpu.VMEM((_RPW,), jnp.float32),
            pltpu.SemaphoreType.DMA,
        ],
    )
    def k(idx_hbm, w_hbm, out_hbm, idx_v, vals_v, out_v, sem):
        wid = lax.axis_index("s") * 2 + lax.axis_index("c")
        pltpu.sync_copy(idx_hbm.at[wid], idx_v)
        pltpu.async_copy(w_hbm.at[idx_v], vals_v, sem).wait()
        for g in range(_RPW // 16):
            acc = vals_v[pl.ds(g * 16, 16)]
            for f in range(1, _F):
                acc = acc + vals_v[pl.ds(f * _RPW + g * 16, 16)]
            out_v[pl.ds(g * 16, 16)] = acc
        pltpu.sync_copy(out_v, out_hbm.at[pl.ds(wid * _RPW, _RPW)])

    return k


_sc_kernel = _make_kernel()


def kernel(inputs, w):
    idx = inputs.astype(jnp.int32).T.reshape(_F, _NW, _RPW)
    idx = idx.transpose(1, 0, 2).reshape(_NW, _CHUNK)
    w_flat = lax.reshape(w, (w.shape[0],), dimensions=(1, 0))
    out = _sc_kernel(idx, w_flat)
    return out.reshape(_B, 1)


# speedup vs baseline: 2.3803x; 2.0909x over previous
import functools

import jax
import jax.numpy as jnp
from jax import lax
from jax.experimental import pallas as pl
from jax.experimental.pallas import tpu as pltpu
from jax.experimental.pallas import tpu_sc as plsc

_B = 16384
_F = 26
_NW = 32
_RPW = _B // _NW
_CHUNK = _F * _RPW


def _make_kernel():
    mesh = plsc.VectorSubcoreMesh(core_axis_name="c", subcore_axis_name="s")

    @functools.partial(
        pl.kernel,
        mesh=mesh,
        out_type=jax.ShapeDtypeStruct((_B,), jnp.float32),
        compiler_params=pltpu.CompilerParams(needs_layout_passes=False),
        scratch_types=[
            pltpu.VMEM((_CHUNK,), jnp.int32),
            pltpu.VMEM((_CHUNK,), jnp.float32),
            pltpu.VMEM((_RPW,), jnp.float32),
            pltpu.SemaphoreType.DMA,
        ],
    )
    def k(idx_hbm, w_hbm, out_hbm, idx_v, vals_v, out_v, sem):
        wid = lax.axis_index("s") * 2 + lax.axis_index("c")
        pltpu.sync_copy(idx_hbm.at[wid], idx_v)
        pltpu.async_copy(w_hbm.at[0].at[idx_v], vals_v, sem).wait()
        for g in range(_RPW // 16):
            acc = vals_v[pl.ds(g * 16, 16)]
            for f in range(1, _F):
                acc = acc + vals_v[pl.ds(f * _RPW + g * 16, 16)]
            out_v[pl.ds(g * 16, 16)] = acc
        pltpu.sync_copy(out_v, out_hbm.at[pl.ds(wid * _RPW, _RPW)])

    return k


_sc_kernel = _make_kernel()


def kernel(inputs, w):
    idx = inputs.astype(jnp.int32).T.reshape(_F, _NW, _RPW)
    idx = idx.transpose(1, 0, 2).reshape(_NW, _CHUNK)
    out = _sc_kernel(idx, w.T)
    return out.reshape(_B, 1)
